# baseline (speedup 1.0000x reference)
import functools

import jax
import jax.numpy as jnp
from jax import lax
from jax.experimental import pallas as pl
from jax.experimental.pallas import tpu as pltpu
from jax.experimental.pallas import tpu_sc as plsc

N_ROWS = 16384
NC = 2
NS = 16
L = 16
NW = NC * NS
ROWS_W = N_ROWS // NW
ELEMS_W = ROWS_W * 4
STEPS = ROWS_W // L


def _sc_body(o_hbm, t_hbm, l_hbm, sums_hbm, cnts_hbm,
             o_v, t_v, l_v, acc_v, cnt_v, sem):
    cid = lax.axis_index("c")
    sid = lax.axis_index("s")
    w = sid * NC + cid
    base_r = w * ROWS_W
    base_e = w * ELEMS_W

    copies = [
        pltpu.async_copy(l_hbm.at[pl.ds(base_r, ROWS_W)], l_v, sem),
        pltpu.async_copy(o_hbm.at[pl.ds(base_e, ELEMS_W)], o_v, sem),
        pltpu.async_copy(t_hbm.at[pl.ds(base_e, ELEMS_W)], t_v, sem),
    ]
    for cp in copies:
        cp.wait()

    ones = jnp.ones((L,), jnp.float32)
    zeros = jnp.zeros((L,), jnp.float32)
    lane4 = lax.shift_right_logical(lax.iota(jnp.int32, L), 2)
    exp_idx = [lane4 + (4 * k) for k in range(4)]
    _dn = lax.GatherDimensionNumbers(
        offset_dims=(), collapsed_slice_dims=(0,), start_index_map=(0,))

    def _take16(vec, idx):
        return lax.gather(vec, idx[:, None], _dn, (1,),
                          mode=lax.GatherScatterMode.PROMISE_IN_BOUNDS)

    def step(i, carry):
        a0, a1, a2, a3, cnt = carry
        lab = l_v[pl.ds(i * L, L)]
        validf = jnp.where(jnp.abs(lab) == 1, ones, zeros)
        base = i * (4 * L)
        m0 = _take16(validf, exp_idx[0])
        m1 = _take16(validf, exp_idx[1])
        m2 = _take16(validf, exp_idx[2])
        m3 = _take16(validf, exp_idx[3])
        d0 = o_v[pl.ds(base, L)] - t_v[pl.ds(base, L)]
        d1 = o_v[pl.ds(base + L, L)] - t_v[pl.ds(base + L, L)]
        d2 = o_v[pl.ds(base + 2 * L, L)] - t_v[pl.ds(base + 2 * L, L)]
        d3 = o_v[pl.ds(base + 3 * L, L)] - t_v[pl.ds(base + 3 * L, L)]
        return (a0 + m0 * (d0 * d0),
                a1 + m1 * (d1 * d1),
                a2 + m2 * (d2 * d2),
                a3 + m3 * (d3 * d3),
                cnt + validf)

    a0, a1, a2, a3, cnt = lax.fori_loop(
        0, STEPS, step, (zeros, zeros, zeros, zeros, zeros))
    acc_v[...] = (a0 + a1) + (a2 + a3)
    cnt_v[...] = cnt
    pltpu.sync_copy(acc_v, sums_hbm.at[pl.ds(w * L, L)])
    pltpu.sync_copy(cnt_v, cnts_hbm.at[pl.ds(w * L, L)])


_sc_call = functools.partial(
    pl.kernel,
    out_type=(jax.ShapeDtypeStruct((NW * L,), jnp.float32),
              jax.ShapeDtypeStruct((NW * L,), jnp.float32)),
    mesh=plsc.VectorSubcoreMesh(core_axis_name="c", subcore_axis_name="s",
                                num_cores=NC, num_subcores=NS),
    scratch_types=[
        pltpu.VMEM((ELEMS_W,), jnp.float32),
        pltpu.VMEM((ELEMS_W,), jnp.float32),
        pltpu.VMEM((ROWS_W,), jnp.int32),
        pltpu.VMEM((L,), jnp.float32),
        pltpu.VMEM((L,), jnp.float32),
        pltpu.SemaphoreType.DMA,
    ],
)(_sc_body)


def kernel(bbox_out, bbox_target, label):
    o = bbox_out.reshape(-1)
    t = bbox_target.reshape(-1)
    sums, cnts = _sc_call(o, t, label)
    total = jnp.sum(sums)
    keep_num = jnp.sum(cnts)
    return total / keep_num
